# baseline (speedup 1.0000x reference)
import jax
import jax.numpy as jnp
from jax.experimental import pallas as pl
from jax.experimental.pallas import tpu as pltpu

ROWS = 128
COLS = 32768
RB = 16
NCHUNK = ROWS // RB
NBUF = 4
AHEAD = 3


def _body(in_ref, out_ref, *scratch):
    bufs = list(scratch[:NBUF])
    sems = scratch[NBUF]

    def copy(k):
        return pltpu.make_async_copy(
            in_ref.at[pl.ds(k * RB, RB)], bufs[k % NBUF], sems.at[k % NBUF]
        )

    copy(0).start()
    started = 1

    idxs = []
    for k in range(NCHUNK):
        copy(k).wait()
        while started < min(k + 1 + AHEAD, NCHUNK):
            copy(started).start()
            started += 1
        a = jnp.argmax(bufs[k % NBUF][...], axis=1)
        idxs.append(a.reshape(RB, 1).astype(jnp.float32))

    idx_f = jnp.concatenate(idxs, axis=0)
    out_ref[...] = jnp.transpose(idx_f).astype(jnp.int32)


def kernel(inputs):
    out = pl.pallas_call(
        _body,
        in_specs=[pl.BlockSpec(memory_space=pl.ANY)],
        out_specs=pl.BlockSpec(memory_space=pltpu.VMEM),
        out_shape=jax.ShapeDtypeStruct((1, ROWS), jnp.int32),
        scratch_shapes=[pltpu.VMEM((RB, COLS), jnp.float32)] * NBUF
        + [pltpu.SemaphoreType.DMA((NBUF,))],
    )(inputs)
    return out.reshape(ROWS)


# speedup vs baseline: 1.0344x; 1.0344x over previous
import jax
import jax.numpy as jnp
from jax.experimental import pallas as pl
from jax.experimental.pallas import tpu as pltpu

ROWS = 128
COLS = 32768
CHUNKS = (4, 4, 4, 4, 16, 16, 16, 16, 16, 16, 16)
assert sum(CHUNKS) == ROWS
OFFS = [sum(CHUNKS[:i]) for i in range(len(CHUNKS))]
PRIME = 5
AHEAD = 4


def _body(in_ref, out_ref, *scratch):
    n = len(CHUNKS)
    bufs = list(scratch[:n])
    sems = scratch[n]

    def copy(k):
        return pltpu.make_async_copy(
            in_ref.at[pl.ds(OFFS[k], CHUNKS[k])], bufs[k], sems.at[k]
        )

    for k in range(PRIME):
        copy(k).start()
    started = PRIME

    idxs = []
    for k in range(n):
        copy(k).wait()
        while started < min(k + 1 + AHEAD, n):
            copy(started).start()
            started += 1
        a = jnp.argmax(bufs[k][...], axis=1)
        idxs.append(a.reshape(CHUNKS[k], 1).astype(jnp.float32))

    idx_f = jnp.concatenate(idxs, axis=0)
    out_ref[...] = jnp.transpose(idx_f).astype(jnp.int32)


def kernel(inputs):
    out = pl.pallas_call(
        _body,
        in_specs=[pl.BlockSpec(memory_space=pl.ANY)],
        out_specs=pl.BlockSpec(memory_space=pltpu.VMEM),
        out_shape=jax.ShapeDtypeStruct((1, ROWS), jnp.int32),
        scratch_shapes=[pltpu.VMEM((rb, COLS), jnp.float32) for rb in CHUNKS]
        + [pltpu.SemaphoreType.DMA((len(CHUNKS),))],
    )(inputs)
    return out.reshape(ROWS)
